# baseline (speedup 1.0000x reference)
import math

import jax
import jax.numpy as jnp
from jax import lax
from jax.experimental import pallas as pl
from jax.experimental.pallas import tpu as pltpu
from jax.experimental.pallas import tpu_sc as plsc

NC = 2
NS = 16
L = 16
NW = NC * NS

B = 4
T = 8192
D = 768
SCALE = math.sqrt(float(D))

TPW = T // NW
CHW = 32
CH = 16
NTC = TPW // CHW
VPR = D // L


def _emb_kernel(ids_hbm, tok_hbm, pos_hbm, out_hbm,
                idx_v, tok0, tok1, tok2, tok3, posbuf, spm,
                gs0, gs1, gs2, gs3, ps, xs00, xs01, xs10, xs11, os0, os1):
    cid = lax.axis_index("c")
    sid = lax.axis_index("s")
    wid = sid * NC + cid
    t0 = wid * TPW

    for b in range(B):
        pltpu.sync_copy(ids_hbm.at[pl.ds(b * T + t0, TPW)],
                        idx_v.at[pl.ds(b * TPW, TPW)])

    toks = (tok0, tok1, tok2, tok3)
    gsems = (gs0, gs1, gs2, gs3)
    xsems = ((xs00, xs01), (xs10, xs11))
    osems = (os0, os1)

    def gather(b_t, tc_t, h_t, slot):
        pltpu.async_copy(
            tok_hbm.at[idx_v.at[pl.ds(b_t * TPW + tc_t * CHW + h_t * CH,
                                      CH)]],
            toks[slot], gsems[slot])

    def wait_xbar(tok_slot, p_s, h_s):
        pltpu.make_async_copy(toks[tok_slot],
                              spm.at[sid, p_s, pl.ds(h_s * CH, CH)],
                              xsems[p_s][h_s]).wait()

    def drain_out(p_s):
        pltpu.make_async_copy(spm.at[sid, p_s],
                              out_hbm.at[pl.ds(0, CHW)],
                              osems[p_s]).wait()

    pltpu.async_copy(pos_hbm.at[pl.ds(t0, CHW)], posbuf, ps)
    gather(0, 0, 0, 0)
    gather(0, 0, 1, 1)

    @pl.loop(0, NTC)
    def _tc(tc):
        for b in range(B):
            p = b % 2
            q = 1 - p
            for h in range(2):
                u = (2 * b + h) % 4
                w = (u + 2) % 4

                if b < B - 1:
                    gather(b + 1, tc, h, w)
                else:
                    @pl.when(tc < NTC - 1)
                    def _():
                        gather(0, tc + 1, h, w)

                pltpu.make_async_copy(
                    tok_hbm.at[pl.ds(0, CH)], toks[u], gsems[u]).wait()
                if b == 0 and h == 0:
                    pltpu.make_async_copy(
                        pos_hbm.at[pl.ds(0, CHW)], posbuf, ps).wait()

                tbuf = toks[u]
                pbase = h * CH

                @pl.loop(0, CH)
                def _row(r):
                    for k in range(VPR):
                        sl = pl.ds(k * L, L)
                        tbuf[r, sl] = (tbuf[r, sl] * SCALE
                                       + posbuf[pbase + r, sl])

                if h == 0:
                    def ship_prev():
                        wait_xbar((2 * b + 3) % 4, q, 1)
                        pltpu.async_copy(
                            spm.at[sid, q],
                            out_hbm.at[pl.ds(
                                ((b - 1) % B) * T + t0
                                + (tc - (1 if b == 0 else 0)) * CHW,
                                CHW)],
                            osems[q])

                    if b == 0:
                        @pl.when(tc > 0)
                        def _():
                            ship_prev()
                    else:
                        ship_prev()

                    if b >= 2:
                        drain_out(p)
                    else:
                        @pl.when(tc > 0)
                        def _():
                            drain_out(p)
                else:
                    wait_xbar((2 * b) % 4, p, 0)

                pltpu.async_copy(tbuf, spm.at[sid, p, pl.ds(pbase, CH)],
                                 xsems[p][h])

            if b == B - 1:
                @pl.when(tc < NTC - 1)
                def _():
                    pltpu.async_copy(
                        pos_hbm.at[pl.ds(t0 + (tc + 1) * CHW, CHW)],
                        posbuf, ps)

    wait_xbar(3, 1, 1)
    pltpu.async_copy(spm.at[sid, 1],
                     out_hbm.at[pl.ds(3 * T + t0 + (NTC - 1) * CHW, CHW)],
                     osems[1])
    drain_out(0)
    drain_out(1)


@jax.jit
def _emb_call(ids_flat, token_embed, pos_embed):
    mesh = plsc.VectorSubcoreMesh(core_axis_name="c", subcore_axis_name="s")
    fn = pl.kernel(
        _emb_kernel,
        out_type=jax.ShapeDtypeStruct((B * T, D), jnp.float32),
        mesh=mesh,
        scratch_types=[
            pltpu.VMEM((B * TPW,), jnp.int32),
            pltpu.VMEM((CH, D), jnp.float32),
            pltpu.VMEM((CH, D), jnp.float32),
            pltpu.VMEM((CH, D), jnp.float32),
            pltpu.VMEM((CH, D), jnp.float32),
            pltpu.VMEM((CHW, D), jnp.float32),
            pltpu.VMEM_SHARED((NS, 2, CHW, D), jnp.float32),
            pltpu.SemaphoreType.DMA,
            pltpu.SemaphoreType.DMA,
            pltpu.SemaphoreType.DMA,
            pltpu.SemaphoreType.DMA,
            pltpu.SemaphoreType.DMA,
            pltpu.SemaphoreType.DMA,
            pltpu.SemaphoreType.DMA,
            pltpu.SemaphoreType.DMA,
            pltpu.SemaphoreType.DMA,
            pltpu.SemaphoreType.DMA,
            pltpu.SemaphoreType.DMA,
        ],
    )
    return fn(ids_flat, token_embed, pos_embed)


def kernel(input_ids, token_embed, pos_embed):
    ids_flat = input_ids.astype(jnp.int32).reshape(B * T)
    out = _emb_call(ids_flat, token_embed, pos_embed)
    return out.reshape(B, T, D)


# speedup vs baseline: 1.1733x; 1.1733x over previous
import math

import jax
import jax.numpy as jnp
from jax import lax
from jax.experimental import pallas as pl
from jax.experimental.pallas import tpu as pltpu
from jax.experimental.pallas import tpu_sc as plsc

NC = 2
NS = 16
L = 16
NW = NC * NS

B = 4
T = 8192
D = 768
SCALE = math.sqrt(float(D))

TPW = T // NW
CH = 16
NTC = TPW // CH
VPR = D // L


def _emb_kernel(ids_hbm, tok_hbm, pos_hbm, out_hbm,
                idx_v, tok0, tok1, tok2, tok3, posbuf, spm,
                gs0, gs1, gs2, gs3, ps, xs0, xs1, os0, os1):
    cid = lax.axis_index("c")
    sid = lax.axis_index("s")
    wid = sid * NC + cid
    t0 = wid * TPW

    for b in range(B):
        pltpu.sync_copy(ids_hbm.at[pl.ds(b * T + t0, TPW)],
                        idx_v.at[pl.ds(b * TPW, TPW)])

    toks = (tok0, tok1, tok2, tok3)
    gsems = (gs0, gs1, gs2, gs3)
    xsems = (xs0, xs1)
    osems = (os0, os1)

    def out_rows(s_tc, s_b):
        return out_hbm.at[pl.ds(s_b * T + t0 + s_tc * CH, CH)]

    def wait_xbar(tok_slot, spm_slot):
        pltpu.make_async_copy(toks[tok_slot], spm.at[sid, spm_slot],
                              xsems[spm_slot]).wait()

    def drain_out(spm_slot):
        pltpu.make_async_copy(spm.at[sid, spm_slot],
                              out_hbm.at[pl.ds(0, CH)],
                              osems[spm_slot]).wait()

    pltpu.async_copy(pos_hbm.at[pl.ds(t0, CH)], posbuf, ps)
    pltpu.async_copy(tok_hbm.at[idx_v.at[pl.ds(0, CH)]], tok0, gs0)
    pltpu.async_copy(tok_hbm.at[idx_v.at[pl.ds(TPW, CH)]], tok1, gs1)

    @pl.loop(0, NTC)
    def _tc(tc):
        for b in range(B):
            u = b
            w = (b + 2) % 4
            p = b % 2
            q = 1 - p

            if b < 2:
                pltpu.async_copy(
                    tok_hbm.at[idx_v.at[pl.ds((b + 2) * TPW + tc * CH,
                                              CH)]],
                    toks[w], gsems[w])
            else:
                @pl.when(tc < NTC - 1)
                def _():
                    pltpu.async_copy(
                        tok_hbm.at[idx_v.at[pl.ds(
                            (b - 2) * TPW + (tc + 1) * CH, CH)]],
                        toks[w], gsems[w])

            pltpu.make_async_copy(
                tok_hbm.at[pl.ds(0, CH)], toks[u], gsems[u]).wait()
            if b == 0:
                pltpu.make_async_copy(
                    pos_hbm.at[pl.ds(0, CH)], posbuf, ps).wait()

            tbuf = toks[u]

            @pl.loop(0, CH)
            def _row(r):
                for k in range(VPR):
                    sl = pl.ds(k * L, L)
                    tbuf[r, sl] = tbuf[r, sl] * SCALE + posbuf[r, sl]

            def ship_prev():
                wait_xbar((b - 1) % 4, q)
                pltpu.async_copy(
                    spm.at[sid, q],
                    out_rows(tc - 1, B - 1) if b == 0
                    else out_rows(tc, b - 1),
                    osems[q])

            if b == 0:
                @pl.when(tc > 0)
                def _():
                    ship_prev()
            else:
                ship_prev()

            if b >= 2:
                drain_out(p)
            else:
                @pl.when(tc > 0)
                def _():
                    drain_out(p)

            pltpu.async_copy(tbuf, spm.at[sid, p], xsems[p])

            if b == B - 1:
                @pl.when(tc < NTC - 1)
                def _():
                    pltpu.async_copy(
                        pos_hbm.at[pl.ds(t0 + (tc + 1) * CH, CH)],
                        posbuf, ps)

    wait_xbar(3, 1)
    drain_out(0)
    pltpu.async_copy(spm.at[sid, 1], out_rows(NTC - 1, 3), osems[1])
    drain_out(1)


@jax.jit
def _emb_call(ids_flat, token_embed, pos_embed):
    mesh = plsc.VectorSubcoreMesh(core_axis_name="c", subcore_axis_name="s")
    fn = pl.kernel(
        _emb_kernel,
        out_type=jax.ShapeDtypeStruct((B * T, D), jnp.float32),
        mesh=mesh,
        scratch_types=[
            pltpu.VMEM((B * TPW,), jnp.int32),
            pltpu.VMEM((CH, D), jnp.float32),
            pltpu.VMEM((CH, D), jnp.float32),
            pltpu.VMEM((CH, D), jnp.float32),
            pltpu.VMEM((CH, D), jnp.float32),
            pltpu.VMEM((CH, D), jnp.float32),
            pltpu.VMEM_SHARED((NS, 2, CH, D), jnp.float32),
            pltpu.SemaphoreType.DMA,
            pltpu.SemaphoreType.DMA,
            pltpu.SemaphoreType.DMA,
            pltpu.SemaphoreType.DMA,
            pltpu.SemaphoreType.DMA,
            pltpu.SemaphoreType.DMA,
            pltpu.SemaphoreType.DMA,
            pltpu.SemaphoreType.DMA,
            pltpu.SemaphoreType.DMA,
        ],
    )
    return fn(ids_flat, token_embed, pos_embed)


def kernel(input_ids, token_embed, pos_embed):
    ids_flat = input_ids.astype(jnp.int32).reshape(B * T)
    out = _emb_call(ids_flat, token_embed, pos_embed)
    return out.reshape(B, T, D)
